# Initial kernel scaffold; baseline (speedup 1.0000x reference)
import jax
import jax.numpy as jnp
from jax.experimental import pallas as pl


def kernel(H, W, HA, WA, hcheb_w, hcheb_b, wcheb_w, wcheb_b, lstm_wih, lstm_whh, lstm_bih, lstm_bhh, dh_w, dh_b, dw_w, dw_b):
    raise NotImplementedError("write your pallas kernel here")


# speedup vs baseline: 1.7955x; 1.7955x over previous
import jax
import jax.numpy as jnp
from jax import lax
from jax.experimental import pallas as pl
from jax.experimental.pallas import tpu as pltpu
from jax.experimental.pallas import tpu_sc as plsc

MM = 10000
EE = 160000
RR = 10
QQ = 32
KK = 5
NHH = 32

RP = 16
NT = 16
NP = 10112
TRASH = MM
ROWS_PT = NP // NT
EPT = EE // NT
CH = 128
NCH = 79
EPTP = NCH * CH


def _sc_body(xin, rows, cols, tx, xs,
             deg_sh, y_sh, row_v, rowa_v, col_v, buf, dis_v, a_v, tm1_v, tm2_v,
             sem):
    g = lax.axis_index("c")
    s = lax.axis_index("s")
    widx = g * NT + s
    base = s * ROWS_PT
    xbase = g * NP + base

    pltpu.sync_copy(rows.at[widx], row_v)
    pltpu.sync_copy(cols.at[widx], col_v)

    goff = g * NP

    def adj(j, _):
        for l in range(CH // 16):
            rowa_v[j, pl.ds(l * 16, 16)] = row_v[j, pl.ds(l * 16, 16)] + goff
        return 0
    lax.fori_loop(0, NCH, adj, 0)

    def zr(r, _):
        a_v[r] = jnp.zeros((16,), jnp.float32)
        return 0
    lax.fori_loop(0, ROWS_PT, zr, 0)
    pltpu.sync_copy(a_v, deg_sh.at[pl.ds(base, ROWS_PT)])

    def on(r, _):
        buf[r] = jnp.ones((16,), jnp.float32)
        return 0
    lax.fori_loop(0, CH, on, 0)
    plsc.subcore_barrier()

    def dg(j, _):
        pltpu.sync_copy(buf, deg_sh.at[row_v.at[j]], add=True)
        return 0
    lax.fori_loop(0, NCH, dg, 0)
    plsc.subcore_barrier()

    pltpu.sync_copy(deg_sh.at[pl.ds(base, ROWS_PT)], a_v)
    pltpu.sync_copy(xin.at[pl.ds(xbase, ROWS_PT)], tm2_v)

    def dz(r, _):
        d = a_v[r]
        xc = jnp.maximum(d, 1.0)
        i = lax.bitcast_convert_type(xc, jnp.int32)
        i = jnp.int32(0x5F3759DF) - (i >> 1)
        y = lax.bitcast_convert_type(i, jnp.float32)
        for _it in range(3):
            y = y * (1.5 - 0.5 * xc * y * y)
        dis = jnp.where(d >= 0.5, y, 0.0)
        dis_v[r] = dis
        a_v[r] = dis * tm2_v[r]
        return 0
    lax.fori_loop(0, ROWS_PT, dz, 0)
    pltpu.sync_copy(a_v, xs.at[pl.ds(xbase, ROWS_PT)])
    plsc.subcore_barrier()

    for k in range(1, KK):
        def zy(r, _):
            a_v[r] = jnp.zeros((16,), jnp.float32)
            return 0
        lax.fori_loop(0, ROWS_PT, zy, 0)
        pltpu.sync_copy(a_v, y_sh.at[pl.ds(base, ROWS_PT)])
        plsc.subcore_barrier()

        def edge(j, _):
            pltpu.async_copy(xs.at[rowa_v.at[j]], buf, sem).wait()
            pltpu.sync_copy(buf, y_sh.at[col_v.at[j]], add=True)
            return 0
        lax.fori_loop(0, NCH, edge, 0)
        plsc.subcore_barrier()

        pltpu.sync_copy(y_sh.at[pl.ds(base, ROWS_PT)], a_v)
        hist = tm1_v if (k % 2 == 1) else tm2_v

        def mg(r, _):
            p = -(dis_v[r] * a_v[r])
            if k == 1:
                t = p
            else:
                t = 2.0 * p - hist[r]
            hist[r] = t
            a_v[r] = dis_v[r] * t
            return 0
        lax.fori_loop(0, ROWS_PT, mg, 0)
        pltpu.sync_copy(hist, tx.at[g * 4 + (k - 1), pl.ds(base, ROWS_PT)])
        if k < KK - 1:
            pltpu.sync_copy(a_v, xs.at[pl.ds(xbase, ROWS_PT)])
        plsc.subcore_barrier()


def _sc_cheb(x2, rows32, cols32):
    mesh = plsc.VectorSubcoreMesh(core_axis_name="c", subcore_axis_name="s")
    f = pl.kernel(
        _sc_body,
        mesh=mesh,
        compiler_params=pltpu.CompilerParams(use_tc_tiling_on_sc=False),
        out_type=[
            jax.ShapeDtypeStruct((8, NP, RP), jnp.float32),
            jax.ShapeDtypeStruct((2 * NP, RP), jnp.float32),
        ],
        scratch_types=[
            pltpu.VMEM_SHARED((NP, RP), jnp.float32),
            pltpu.VMEM_SHARED((NP, RP), jnp.float32),
            pltpu.VMEM((NCH, CH), jnp.int32),
            pltpu.VMEM((NCH, CH), jnp.int32),
            pltpu.VMEM((NCH, CH), jnp.int32),
            pltpu.VMEM((CH, RP), jnp.float32),
            pltpu.VMEM((ROWS_PT, RP), jnp.float32),
            pltpu.VMEM((ROWS_PT, RP), jnp.float32),
            pltpu.VMEM((ROWS_PT, RP), jnp.float32),
            pltpu.VMEM((ROWS_PT, RP), jnp.float32),
            pltpu.SemaphoreType.DMA,
        ],
    )
    return f(x2, rows32, cols32)


def _tc_body(x_ref, cw_ref, cb_ref, wihT_ref, whhT_ref, bsum_ref,
             dwT_ref, db_ref, hout_ref, wout_ref,
             xg0_ref, xg1_ref, o3_ref, o4_ref):
    whhT = whhT_ref[...]
    bsum = bsum_ref[...]
    for gi in range(2):
        acc = jnp.dot(x_ref[gi], cw_ref[gi], preferred_element_type=jnp.float32)
        til = jax.nn.sigmoid(acc + cb_ref[gi])
        xg = jnp.dot(til, wihT_ref[...], preferred_element_type=jnp.float32) + bsum
        if gi == 0:
            xg0_ref[...] = xg
        else:
            xg1_ref[...] = xg

    def make_step(xg_ref, out_ref):
        def step(t, carry):
            h, c = carry
            gg = xg_ref[pl.ds(t, 1), :] + jnp.dot(h, whhT,
                                                  preferred_element_type=jnp.float32)
            i = jax.nn.sigmoid(gg[:, 0:NHH])
            fo = jax.nn.sigmoid(gg[:, NHH:2 * NHH])
            u = jnp.tanh(gg[:, 2 * NHH:3 * NHH])
            o = jax.nn.sigmoid(gg[:, 3 * NHH:4 * NHH])
            c = fo * c + i * u
            h = o * jnp.tanh(c)
            if out_ref is not None:
                out_ref[pl.ds(t, 1), :] = h
            return (h, c)
        return step

    h = jnp.zeros((1, NHH), jnp.float32)
    c = jnp.zeros((1, NHH), jnp.float32)
    for xgr, outr in ((xg0_ref, None), (xg1_ref, None), (xg0_ref, o3_ref), (xg1_ref, o4_ref)):
        h, c = lax.fori_loop(0, MM, make_step(xgr, outr), (h, c))

    dH = jnp.tanh(jnp.dot(o3_ref[...], dwT_ref[0], preferred_element_type=jnp.float32)
                  + db_ref[0])
    hout_ref[...] = x_ref[0, 0:MM, 0:RR] + dH[:, 0:RR]
    dW = jnp.tanh(jnp.dot(o4_ref[...], dwT_ref[1], preferred_element_type=jnp.float32)
                  + db_ref[1])
    wout_ref[...] = x_ref[1, 0:MM, 0:RR] + dW[:, 0:RR]


def kernel(H, W, HA, WA, hcheb_w, hcheb_b, wcheb_w, wcheb_b,
           lstm_wih, lstm_whh, lstm_bih, lstm_bhh, dh_w, dh_b, dw_w, dw_b):
    f32 = jnp.float32
    Hp = jnp.pad(H.astype(f32), ((0, NP - MM), (0, RP - RR)))
    Wp = jnp.pad(W.astype(f32), ((0, NP - MM), (0, RP - RR)))
    X2 = jnp.concatenate([Hp, Wp], axis=0)

    def edges(A):
        pad = jnp.full((NT, EPTP - EPT), TRASH, jnp.int32)
        r = jnp.concatenate([A[0].astype(jnp.int32).reshape(NT, EPT), pad], axis=1)
        c = jnp.concatenate([A[1].astype(jnp.int32).reshape(NT, EPT), pad], axis=1)
        return r.reshape(NT, NCH, CH), c.reshape(NT, NCH, CH)

    rH, cH = edges(HA)
    rW, cW = edges(WA)
    rows32 = jnp.concatenate([rH, rW], axis=0)
    cols32 = jnp.concatenate([cH, cW], axis=0)

    tx, _xs = _sc_cheb(X2, rows32, cols32)

    cw = jnp.stack([jnp.pad(hcheb_w, ((0, 0), (0, RP - RR), (0, 0))),
                    jnp.pad(wcheb_w, ((0, 0), (0, RP - RR), (0, 0)))])
    cw = cw.reshape(2, KK * RP, QQ)
    cb = jnp.stack([hcheb_b, wcheb_b]).reshape(2, 1, QQ)
    wihT = lstm_wih.T
    whhT = lstm_whh.T
    bsum = (lstm_bih + lstm_bhh).reshape(1, 4 * NHH)
    dwT = jnp.stack([jnp.pad(dh_w.T, ((0, 0), (0, RP - RR))),
                     jnp.pad(dw_w.T, ((0, 0), (0, RP - RR)))])
    db = jnp.stack([jnp.pad(dh_b, (0, RP - RR)),
                    jnp.pad(dw_b, (0, RP - RR))]).reshape(2, 1, RP)

    X3 = X2.reshape(2, NP, RP)
    txg = tx.reshape(2, 4, NP, RP).transpose(0, 2, 1, 3).reshape(2, NP, 4 * RP)
    Xcat = jnp.concatenate([X3, txg], axis=2)
    Hout, Wout = pl.pallas_call(
        _tc_body,
        out_shape=[jax.ShapeDtypeStruct((MM, RR), f32),
                   jax.ShapeDtypeStruct((MM, RR), f32)],
        scratch_shapes=[
            pltpu.VMEM((NP, 4 * NHH), f32),
            pltpu.VMEM((NP, 4 * NHH), f32),
            pltpu.VMEM((MM, NHH), f32),
            pltpu.VMEM((MM, NHH), f32),
        ],
    )(Xcat, cw, cb, wihT, whhT, bsum, dwT, db)
    return (Hout, Wout)


# speedup vs baseline: 3.4712x; 1.9333x over previous
import jax
import jax.numpy as jnp
from jax import lax
from jax.experimental import pallas as pl
from jax.experimental.pallas import tpu as pltpu
from jax.experimental.pallas import tpu_sc as plsc

MM = 10000
EE = 160000
RR = 10
QQ = 32
KK = 5
NHH = 32

RP = 16
NT = 16
NP = 10112
TRASH = MM
ROWS_PT = NP // NT
EPT = EE // NT
CH = 128
NCH = 79
EPTP = NCH * CH


def _sc_body(xin, rows, cols, tx, xs,
             deg_sh, y_sh, row_v, rowa_v, col_v, buf, dis_v, a_v, tm1_v, tm2_v,
             sem):
    g = lax.axis_index("c")
    s = lax.axis_index("s")
    widx = g * NT + s
    base = s * ROWS_PT
    xbase = g * NP + base

    pltpu.sync_copy(rows.at[widx], row_v)
    pltpu.sync_copy(cols.at[widx], col_v)

    goff = g * NP

    def adj(j, _):
        for l in range(CH // 16):
            rowa_v[j, pl.ds(l * 16, 16)] = row_v[j, pl.ds(l * 16, 16)] + goff
        return 0
    lax.fori_loop(0, NCH, adj, 0)

    def zr(r, _):
        a_v[r] = jnp.zeros((16,), jnp.float32)
        return 0
    lax.fori_loop(0, ROWS_PT, zr, 0)
    pltpu.sync_copy(a_v, deg_sh.at[pl.ds(base, ROWS_PT)])

    def on(r, _):
        buf[r] = jnp.ones((16,), jnp.float32)
        return 0
    lax.fori_loop(0, CH, on, 0)
    plsc.subcore_barrier()

    def dg(j, _):
        pltpu.sync_copy(buf, deg_sh.at[row_v.at[j]], add=True)
        return 0
    lax.fori_loop(0, NCH, dg, 0)
    plsc.subcore_barrier()

    pltpu.sync_copy(deg_sh.at[pl.ds(base, ROWS_PT)], a_v)
    pltpu.sync_copy(xin.at[pl.ds(xbase, ROWS_PT)], tm2_v)

    def dz(r, _):
        d = a_v[r]
        xc = jnp.maximum(d, 1.0)
        i = lax.bitcast_convert_type(xc, jnp.int32)
        i = jnp.int32(0x5F3759DF) - (i >> 1)
        y = lax.bitcast_convert_type(i, jnp.float32)
        for _it in range(3):
            y = y * (1.5 - 0.5 * xc * y * y)
        dis = jnp.where(d >= 0.5, y, 0.0)
        dis_v[r] = dis
        a_v[r] = dis * tm2_v[r]
        return 0
    lax.fori_loop(0, ROWS_PT, dz, 0)
    pltpu.sync_copy(a_v, xs.at[pl.ds(xbase, ROWS_PT)])
    plsc.subcore_barrier()

    for k in range(1, KK):
        def zy(r, _):
            a_v[r] = jnp.zeros((16,), jnp.float32)
            return 0
        lax.fori_loop(0, ROWS_PT, zy, 0)
        pltpu.sync_copy(a_v, y_sh.at[pl.ds(base, ROWS_PT)])
        plsc.subcore_barrier()

        def edge(j, _):
            pltpu.async_copy(xs.at[rowa_v.at[j]], buf, sem).wait()
            pltpu.sync_copy(buf, y_sh.at[col_v.at[j]], add=True)
            return 0
        lax.fori_loop(0, NCH, edge, 0)
        plsc.subcore_barrier()

        pltpu.sync_copy(y_sh.at[pl.ds(base, ROWS_PT)], a_v)
        hist = tm1_v if (k % 2 == 1) else tm2_v

        def mg(r, _):
            p = -(dis_v[r] * a_v[r])
            if k == 1:
                t = p
            else:
                t = 2.0 * p - hist[r]
            hist[r] = t
            a_v[r] = dis_v[r] * t
            return 0
        lax.fori_loop(0, ROWS_PT, mg, 0)
        pltpu.sync_copy(hist, tx.at[g * 4 + (k - 1), pl.ds(base, ROWS_PT)])
        if k < KK - 1:
            pltpu.sync_copy(a_v, xs.at[pl.ds(xbase, ROWS_PT)])
        plsc.subcore_barrier()


def _sc_cheb(x2, rows32, cols32):
    mesh = plsc.VectorSubcoreMesh(core_axis_name="c", subcore_axis_name="s")
    f = pl.kernel(
        _sc_body,
        mesh=mesh,
        compiler_params=pltpu.CompilerParams(use_tc_tiling_on_sc=False),
        out_type=[
            jax.ShapeDtypeStruct((8, NP, RP), jnp.float32),
            jax.ShapeDtypeStruct((2 * NP, RP), jnp.float32),
        ],
        scratch_types=[
            pltpu.VMEM_SHARED((NP, RP), jnp.float32),
            pltpu.VMEM_SHARED((NP, RP), jnp.float32),
            pltpu.VMEM((NCH, CH), jnp.int32),
            pltpu.VMEM((NCH, CH), jnp.int32),
            pltpu.VMEM((NCH, CH), jnp.int32),
            pltpu.VMEM((CH, RP), jnp.float32),
            pltpu.VMEM((ROWS_PT, RP), jnp.float32),
            pltpu.VMEM((ROWS_PT, RP), jnp.float32),
            pltpu.VMEM((ROWS_PT, RP), jnp.float32),
            pltpu.VMEM((ROWS_PT, RP), jnp.float32),
            pltpu.SemaphoreType.DMA,
        ],
    )
    return f(x2, rows32, cols32)


def _tc_body(x_ref, cw_ref, cb_ref, wihT_ref, whh128_ref, bsum_ref, gs_ref,
             ga_ref, gb_ref, dwT_ref, db_ref, hout_ref, wout_ref,
             xg0_ref, xg1_ref, o3_ref, o4_ref):
    whh128 = whh128_ref[...]
    bsum = bsum_ref[...]
    gs = gs_ref[...]
    ga = ga_ref[...]
    gb = gb_ref[...]
    for gi in range(2):
        acc = jnp.dot(x_ref[gi], cw_ref[gi], preferred_element_type=jnp.float32)
        til = jax.nn.sigmoid(acc + cb_ref[gi])
        xg = (jnp.dot(til, wihT_ref[...], preferred_element_type=jnp.float32)
              + bsum) * gs
        if gi == 0:
            xg0_ref[...] = xg
        else:
            xg1_ref[...] = xg

    def make_step(xg_ref, out_ref):
        def step(t, carry):
            h, c = carry
            g = xg_ref[pl.ds(t, 1), :] + jnp.dot(h, whh128,
                                                 preferred_element_type=jnp.float32)
            th = jnp.tanh(g)
            gates = ga * th + gb
            i_al = pltpu.roll(gates, NHH, 1)
            u_al = pltpu.roll(gates, 3 * NHH, 1)
            o_al = pltpu.roll(gates, 2 * NHH, 1)
            c = gates * c + i_al * u_al
            h = o_al * jnp.tanh(c)
            if out_ref is not None:
                out_ref[pl.ds(t, 1), :] = h
            return (h, c)
        return step

    h = jnp.zeros((1, 4 * NHH), jnp.float32)
    c = jnp.zeros((1, 4 * NHH), jnp.float32)
    for xgr, outr in ((xg0_ref, None), (xg1_ref, None), (xg0_ref, o3_ref), (xg1_ref, o4_ref)):
        h, c = lax.fori_loop(0, MM, make_step(xgr, outr), (h, c))

    dH = jnp.tanh(jnp.dot(o3_ref[...], dwT_ref[0], preferred_element_type=jnp.float32)
                  + db_ref[0])
    hout_ref[...] = x_ref[0, 0:MM, 0:RR] + dH[:, 0:RR]
    dW = jnp.tanh(jnp.dot(o4_ref[...], dwT_ref[1], preferred_element_type=jnp.float32)
                  + db_ref[1])
    wout_ref[...] = x_ref[1, 0:MM, 0:RR] + dW[:, 0:RR]


def kernel(H, W, HA, WA, hcheb_w, hcheb_b, wcheb_w, wcheb_b,
           lstm_wih, lstm_whh, lstm_bih, lstm_bhh, dh_w, dh_b, dw_w, dw_b):
    f32 = jnp.float32
    Hp = jnp.pad(H.astype(f32), ((0, NP - MM), (0, RP - RR)))
    Wp = jnp.pad(W.astype(f32), ((0, NP - MM), (0, RP - RR)))
    X2 = jnp.concatenate([Hp, Wp], axis=0)

    def edges(A):
        pad = jnp.full((NT, EPTP - EPT), TRASH, jnp.int32)
        r = jnp.concatenate([A[0].astype(jnp.int32).reshape(NT, EPT), pad], axis=1)
        c = jnp.concatenate([A[1].astype(jnp.int32).reshape(NT, EPT), pad], axis=1)
        return r.reshape(NT, NCH, CH), c.reshape(NT, NCH, CH)

    rH, cH = edges(HA)
    rW, cW = edges(WA)
    rows32 = jnp.concatenate([rH, rW], axis=0)
    cols32 = jnp.concatenate([cH, cW], axis=0)

    tx, _xs = _sc_cheb(X2, rows32, cols32)

    cw = jnp.stack([jnp.pad(hcheb_w, ((0, 0), (0, RP - RR), (0, 0))),
                    jnp.pad(wcheb_w, ((0, 0), (0, RP - RR), (0, 0)))])
    cw = cw.reshape(2, KK * RP, QQ)
    cb = jnp.stack([hcheb_b, wcheb_b]).reshape(2, 1, QQ)
    wihT = lstm_wih.T
    bsum = (lstm_bih + lstm_bhh).reshape(1, 4 * NHH)
    half = jnp.ones((1, NHH), f32) * 0.5
    one = jnp.ones((1, NHH), f32)
    zero = jnp.zeros((1, NHH), f32)
    gs = jnp.concatenate([half, half, one, half], axis=1)
    ga = jnp.concatenate([half, half, one, half], axis=1)
    gb = jnp.concatenate([half, half, zero, half], axis=1)
    whh128 = jnp.zeros((4 * NHH, 4 * NHH), f32).at[NHH:2 * NHH, :].set(
        lstm_whh.T * gs)
    dwT = jnp.stack([
        jnp.zeros((4 * NHH, RP), f32).at[NHH:2 * NHH, 0:RR].set(dh_w.T),
        jnp.zeros((4 * NHH, RP), f32).at[NHH:2 * NHH, 0:RR].set(dw_w.T)])
    db = jnp.stack([jnp.pad(dh_b, (0, RP - RR)),
                    jnp.pad(dw_b, (0, RP - RR))]).reshape(2, 1, RP)

    X3 = X2.reshape(2, NP, RP)
    txg = tx.reshape(2, 4, NP, RP).transpose(0, 2, 1, 3).reshape(2, NP, 4 * RP)
    Xcat = jnp.concatenate([X3, txg], axis=2)
    Hout, Wout = pl.pallas_call(
        _tc_body,
        out_shape=[jax.ShapeDtypeStruct((MM, RR), f32),
                   jax.ShapeDtypeStruct((MM, RR), f32)],
        scratch_shapes=[
            pltpu.VMEM((NP, 4 * NHH), f32),
            pltpu.VMEM((NP, 4 * NHH), f32),
            pltpu.VMEM((MM, 4 * NHH), f32),
            pltpu.VMEM((MM, 4 * NHH), f32),
        ],
    )(Xcat, cw, cb, wihT, whh128, bsum, gs, ga, gb, dwT, db)
    return (Hout, Wout)
